# Initial kernel scaffold; baseline (speedup 1.0000x reference)
import jax
import jax.numpy as jnp
from jax.experimental import pallas as pl


def kernel(embedding):
    raise NotImplementedError("write your pallas kernel here")


# speedup vs baseline: 16.1212x; 16.1212x over previous
import functools

import jax
import jax.numpy as jnp
from jax import lax
from jax.experimental import pallas as pl
from jax.experimental.pallas import tpu as pltpu
from jax.experimental.pallas import tpu_sc as plsc

SEQ = 2048
UNITS = 32
ROW_W = SEQ * UNITS

_info = plsc.get_sparse_core_info()
_NC = _info.num_cores
_NS = _info.num_subcores
_NW = _NC * _NS
_RPW = SEQ // _NW

_mesh = plsc.VectorSubcoreMesh(core_axis_name="c", subcore_axis_name="s")


@functools.partial(
    pl.kernel,
    mesh=_mesh,
    out_type=jax.ShapeDtypeStruct((SEQ, ROW_W), jnp.float32),
    scratch_types=[
        pltpu.VMEM((5 * UNITS,), jnp.float32),
        pltpu.VMEM((ROW_W,), jnp.float32),
    ],
)
def _rel_pos_sc(emb_hbm, out_hbm, emb_v, slab_v):
    wid = lax.axis_index("s") * _NC + lax.axis_index("c")
    i0 = wid * _RPW

    pltpu.sync_copy(emb_hbm, emb_v)

    e0a = emb_v[pl.ds(0 * UNITS, 16)]
    e0b = emb_v[pl.ds(0 * UNITS + 16, 16)]
    e1a = emb_v[pl.ds(1 * UNITS, 16)]
    e1b = emb_v[pl.ds(1 * UNITS + 16, 16)]
    e2a = emb_v[pl.ds(2 * UNITS, 16)]
    e2b = emb_v[pl.ds(2 * UNITS + 16, 16)]
    e3a = emb_v[pl.ds(3 * UNITS, 16)]
    e3b = emb_v[pl.ds(3 * UNITS + 16, 16)]
    e4a = emb_v[pl.ds(4 * UNITS, 16)]
    e4b = emb_v[pl.ds(4 * UNITS + 16, 16)]

    def set_row(r, a, b):
        slab_v[pl.ds(r * UNITS, 16)] = a
        slab_v[pl.ds(r * UNITS + 16, 16)] = b

    def fill0(r, _):
        set_row(r, e0a, e0b)
        return 0

    def fill4(r, _):
        set_row(r, e4a, e4b)
        return 0

    lax.fori_loop(0, jnp.maximum(i0 - 1, 0), fill0, 0)

    @pl.when(i0 - 1 >= 0)
    def _():
        set_row(i0 - 1, e1a, e1b)

    set_row(i0, e2a, e2b)

    @pl.when(i0 + 1 < SEQ)
    def _():
        set_row(i0 + 1, e3a, e3b)

    lax.fori_loop(i0 + 2, SEQ, fill4, 0)

    def body(n, _):
        i = i0 + n

        @pl.when(n > 0)
        def _():
            @pl.when(i - 2 >= 0)
            def _():
                set_row(i - 2, e0a, e0b)

            set_row(i - 1, e1a, e1b)
            set_row(i, e2a, e2b)

            @pl.when(i + 1 < SEQ)
            def _():
                set_row(i + 1, e3a, e3b)

        pltpu.sync_copy(slab_v, out_hbm.at[i])
        return 0

    lax.fori_loop(0, _RPW, body, 0)


def kernel(embedding):
    out = _rel_pos_sc(embedding.reshape(5 * UNITS))
    return out.reshape(SEQ, SEQ, UNITS)
